# Initial kernel scaffold; baseline (speedup 1.0000x reference)
import jax
import jax.numpy as jnp
from jax.experimental import pallas as pl


def kernel(node_features, edge_index, edge_norm, edge_type, basis, comp, root_w, root_b, gc_w_rel, gc_w_root, gc_b):
    raise NotImplementedError("write your pallas kernel here")


# speedup vs baseline: 6.5319x; 6.5319x over previous
import functools

import jax
import jax.numpy as jnp
from jax import lax
from jax.experimental import pallas as pl
from jax.experimental.pallas import tpu as pltpu
from jax.experimental.pallas import tpu_sc as plsc

N = 10000
E = 320000
D = 128
R = 8
NB = 30
NC, NS, L = 2, 16, 16
NW = NC * NS
EPT = E // NW
EPS = E // NS
CH = 80
NCH_MSG = EPT // CH
NCH_HIST = EPS // CH
ZR = 24
NH = N // NC
AROWS = 5008
NRA = 312
CNTH = NH * R
CNTP = 40960
MROWS = 1000
GRID = N // MROWS


def _wr_body(comp_ref, basis_ref, out_ref):
    out_ref[...] = jnp.dot(comp_ref[...], basis_ref[...],
                           preferred_element_type=jnp.float32)


def _xw_body(x_ref, w_ref, root_ref, rel_ref):
    y = jnp.dot(x_ref[...], w_ref[...], preferred_element_type=jnp.float32)
    root_ref[...] = y[:, :D]
    rel_ref[...] = y[:, D:]


def _out1_body(root_ref, b_ref, a_ref, o_ref):
    o_ref[...] = root_ref[...] + b_ref[...] + a_ref[...]


def _out2_body(n_ref, o1_ref, wr_ref, wo_ref, b_ref, o_ref):
    o_ref[...] = (jnp.dot(n_ref[...], wr_ref[...], preferred_element_type=jnp.float32)
                  + jnp.dot(o1_ref[...], wo_ref[...], preferred_element_type=jnp.float32)
                  + b_ref[...])


_SC_MESH = plsc.VectorSubcoreMesh(core_axis_name="c", subcore_axis_name="s")


def _zero_accum_slice(acc_sh, zrows, sid, nr, rem):
    def zrow(j, carry):
        for c in range(D // L):
            zrows[j, pl.ds(c * L, L)] = jnp.zeros((L,), jnp.float32)
        return carry
    lax.fori_loop(0, ZR, zrow, 0)

    def zcopy(j, carry):
        pltpu.sync_copy(zrows, acc_sh.at[pl.ds(sid * nr + j * ZR, ZR)])
        return carry
    lax.fori_loop(0, nr // ZR, zcopy, 0)

    @pl.when(sid == NS - 1)
    def _():
        pltpu.sync_copy(zrows.at[pl.ds(0, rem)],
                        acc_sh.at[pl.ds(NS * nr, rem)])


def _dump_accum_slice(acc_sh, zrows, out_ref, cid, sid, nr, rem):
    def dump(j, carry):
        off = sid * nr + j * ZR
        pltpu.sync_copy(acc_sh.at[pl.ds(off, ZR)], zrows)
        pltpu.sync_copy(zrows, out_ref.at[cid, pl.ds(off, ZR)])
        return carry
    lax.fori_loop(0, nr // ZR, dump, 0)

    @pl.when(sid == NS - 1)
    def _():
        pltpu.sync_copy(acc_sh.at[pl.ds(NS * nr, rem)],
                        zrows.at[pl.ds(0, rem)])
        pltpu.sync_copy(zrows.at[pl.ds(0, rem)],
                        out_ref.at[cid, pl.ds(NS * nr, rem)])


def _msg_body(xwrel, srcv, dstv, etv, accout,
              cnt_sh, acc_sh, sbuf, rows, srcb, dstb, etb,
              jrowb, jaccb, svalsb, onesb, zrows, gsem):
    cid = lax.axis_index("c")
    sid = lax.axis_index("s")
    nbase = cid * NH

    cslice = CNTP // NS

    def zs(i, carry):
        sbuf[pl.ds(i * L, L)] = jnp.zeros((L,), jnp.float32)
        return carry
    lax.fori_loop(0, cslice // L, zs, 0)
    pltpu.sync_copy(sbuf.at[pl.ds(0, cslice)],
                    cnt_sh.at[pl.ds(sid * cslice, cslice)])
    _zero_accum_slice(acc_sh, zrows, sid, NRA, AROWS - NS * NRA)

    def set_ones(i, carry):
        onesb[pl.ds(i * L, L)] = jnp.full((L,), 1.0, jnp.float32)
        return carry
    lax.fori_loop(0, CH // L, set_ones, 0)
    plsc.subcore_barrier()

    def hist_chunk(k, carry):
        base = sid * EPS + k * CH
        pltpu.sync_copy(dstv.at[pl.ds(base, CH)], dstb)
        pltpu.sync_copy(etv.at[pl.ds(base, CH)], etb)

        def mk(i, c2):
            rel = dstb[pl.ds(i * L, L)] - nbase
            ok = (rel >= 0) & (rel < NH)
            jv = jnp.where(ok, rel * R + etb[pl.ds(i * L, L)], CNTH)
            jrowb[0, pl.ds(i * L, L)] = jv
            return c2
        lax.fori_loop(0, CH // L, mk, 0)
        pltpu.sync_copy(onesb, cnt_sh.at[jrowb.at[0]], add=True)
        return carry
    lax.fori_loop(0, NCH_HIST, hist_chunk, 0)
    plsc.subcore_barrier()

    pltpu.sync_copy(cnt_sh, sbuf)

    def mks(i, carry):
        v = sbuf[pl.ds(i * L, L)]
        sbuf[pl.ds(i * L, L)] = 1.0 / jnp.maximum(v, 1.0)
        return carry
    lax.fori_loop(0, CNTP // L, mks, 0)

    def msg_chunk(k, carry):
        base = sid * EPS + k * CH
        pltpu.sync_copy(srcv.at[pl.ds(base, CH)], srcb)
        pltpu.sync_copy(dstv.at[pl.ds(base, CH)], dstb)
        pltpu.sync_copy(etv.at[pl.ds(base, CH)], etb)

        def mk2(i, c2):
            sv = srcb[pl.ds(i * L, L)]
            tv = etb[pl.ds(i * L, L)]
            rel = dstb[pl.ds(i * L, L)] - nbase
            ok = (rel >= 0) & (rel < NH)
            jrowb[0, pl.ds(i * L, L)] = sv * R + tv
            jaccb[0, pl.ds(i * L, L)] = jnp.where(ok, rel, NH)
            jc = jnp.where(ok, rel * R + tv, CNTH)
            svalsb[pl.ds(i * L, L)] = plsc.load_gather(sbuf, [jc])
            return c2
        lax.fori_loop(0, CH // L, mk2, 0)

        cp = pltpu.async_copy(xwrel.at[jrowb.at[0]], rows, gsem)
        cp.wait()

        def scale_grp(g, c2):
            for le in range(L):
                e = g * L + le
                ev = jnp.zeros((L,), jnp.int32) + e
                spl = plsc.load_gather(svalsb, [ev])
                for c in range(D // L):
                    cv = lax.iota(jnp.int32, L) + c * L
                    v = plsc.load_gather(rows, [ev, cv])
                    plsc.store_scatter(rows, [ev, cv], v * spl)
            return c2
        lax.fori_loop(0, CH // L, scale_grp, 0)

        pltpu.sync_copy(rows, acc_sh.at[jaccb.at[0]], add=True)
        return carry
    lax.fori_loop(0, NCH_HIST, msg_chunk, 0)
    plsc.subcore_barrier()

    _dump_accum_slice(acc_sh, zrows, accout, cid, sid, NRA, AROWS - NS * NRA)


def _agg_body(o1, srcv, dstv, nout, acc_sh, rows, srcb, dstb, jaccb, zrows,
              gsem):
    cid = lax.axis_index("c")
    sid = lax.axis_index("s")
    nbase = cid * NH

    _zero_accum_slice(acc_sh, zrows, sid, NRA, AROWS - NS * NRA)
    plsc.subcore_barrier()

    def chunk(k, carry):
        base = sid * EPS + k * CH
        pltpu.sync_copy(srcv.at[pl.ds(base, CH)], srcb)
        pltpu.sync_copy(dstv.at[pl.ds(base, CH)], dstb)

        def mk(i, c2):
            dv = dstb[pl.ds(i * L, L)]
            rel = dv - nbase
            ok = (rel >= 0) & (rel < NH)
            jaccb[0, pl.ds(i * L, L)] = jnp.where(ok, rel, NH)
            return c2
        lax.fori_loop(0, CH // L, mk, 0)

        pltpu.async_copy(o1.at[srcb], rows, gsem).wait()
        pltpu.sync_copy(rows, acc_sh.at[jaccb.at[0]], add=True)
        return carry
    lax.fori_loop(0, NCH_HIST, chunk, 0)
    plsc.subcore_barrier()

    _dump_accum_slice(acc_sh, zrows, nout, cid, sid, NRA, AROWS - NS * NRA)


_msg_call = pl.kernel(
    _msg_body,
    out_type=jax.ShapeDtypeStruct((NC, AROWS, D), jnp.float32),
    mesh=_SC_MESH,
    compiler_params=pltpu.CompilerParams(needs_layout_passes=False),
    scratch_types=[
        pltpu.VMEM_SHARED((CNTP,), jnp.float32),
        pltpu.VMEM_SHARED((AROWS, D), jnp.float32),
        pltpu.VMEM((CNTP,), jnp.float32),
        pltpu.VMEM((CH, D), jnp.float32),
        pltpu.VMEM((CH,), jnp.int32),
        pltpu.VMEM((CH,), jnp.int32),
        pltpu.VMEM((CH,), jnp.int32),
        pltpu.VMEM((1, CH), jnp.int32),
        pltpu.VMEM((1, CH), jnp.int32),
        pltpu.VMEM((CH,), jnp.float32),
        pltpu.VMEM((CH,), jnp.float32),
        pltpu.VMEM((ZR, D), jnp.float32),
        pltpu.SemaphoreType.DMA,
    ],
)

_agg_call = pl.kernel(
    _agg_body,
    out_type=jax.ShapeDtypeStruct((NC, AROWS, D), jnp.float32),
    mesh=_SC_MESH,
    compiler_params=pltpu.CompilerParams(needs_layout_passes=False),
    scratch_types=[
        pltpu.VMEM_SHARED((AROWS, D), jnp.float32),
        pltpu.VMEM((CH, D), jnp.float32),
        pltpu.VMEM((CH,), jnp.int32),
        pltpu.VMEM((CH,), jnp.int32),
        pltpu.VMEM((1, CH), jnp.int32),
        pltpu.VMEM((ZR, D), jnp.float32),
        pltpu.SemaphoreType.DMA,
    ],
)


def kernel(node_features, edge_index, edge_norm, edge_type, basis, comp,
           root_w, root_b, gc_w_rel, gc_w_root, gc_b):
    del edge_norm
    x = node_features
    src = edge_index[0].astype(jnp.int32)
    dst = edge_index[1].astype(jnp.int32)
    et = edge_type.astype(jnp.int32)

    basis2d = basis.reshape(NB, D * D)
    wr_flat = pl.pallas_call(
        _wr_body,
        out_shape=jax.ShapeDtypeStruct((R, D * D), jnp.float32),
    )(comp, basis2d)
    w_all = wr_flat.reshape(R, D, D).transpose(1, 0, 2).reshape(D, R * D)
    wfull = jnp.concatenate([root_w, w_all], axis=1)

    root_part, rel_part = pl.pallas_call(
        _xw_body,
        grid=(GRID,),
        in_specs=[
            pl.BlockSpec((MROWS, D), lambda i: (i, 0)),
            pl.BlockSpec((D, (R + 1) * D), lambda i: (0, 0)),
        ],
        out_specs=[
            pl.BlockSpec((MROWS, D), lambda i: (i, 0)),
            pl.BlockSpec((MROWS, R * D), lambda i: (i, 0)),
        ],
        out_shape=[
            jax.ShapeDtypeStruct((N, D), jnp.float32),
            jax.ShapeDtypeStruct((N, R * D), jnp.float32),
        ],
    )(x, wfull)
    xwrel = rel_part.reshape(N * R, D)

    acc = _msg_call(xwrel, src, dst, et)
    accfull = jnp.concatenate([acc[0, :NH], acc[1, :NH]], axis=0)

    out1 = pl.pallas_call(
        _out1_body,
        grid=(GRID,),
        in_specs=[
            pl.BlockSpec((MROWS, D), lambda i: (i, 0)),
            pl.BlockSpec((1, D), lambda i: (0, 0)),
            pl.BlockSpec((MROWS, D), lambda i: (i, 0)),
        ],
        out_specs=pl.BlockSpec((MROWS, D), lambda i: (i, 0)),
        out_shape=jax.ShapeDtypeStruct((N, D), jnp.float32),
    )(root_part, root_b.reshape(1, D), accfull)

    nacc = _agg_call(out1, src, dst)
    neigh = jnp.concatenate([nacc[0, :NH], nacc[1, :NH]], axis=0)

    out2 = pl.pallas_call(
        _out2_body,
        grid=(GRID,),
        in_specs=[
            pl.BlockSpec((MROWS, D), lambda i: (i, 0)),
            pl.BlockSpec((MROWS, D), lambda i: (i, 0)),
            pl.BlockSpec((D, D), lambda i: (0, 0)),
            pl.BlockSpec((D, D), lambda i: (0, 0)),
            pl.BlockSpec((1, D), lambda i: (0, 0)),
        ],
        out_specs=pl.BlockSpec((MROWS, D), lambda i: (i, 0)),
        out_shape=jax.ShapeDtypeStruct((N, D), jnp.float32),
    )(neigh, out1, gc_w_rel, gc_w_root, gc_b.reshape(1, D))
    return out2
